# Initial kernel scaffold; baseline (speedup 1.0000x reference)
import jax
import jax.numpy as jnp
from jax.experimental import pallas as pl


def kernel(Fa, Fb, a_idx, b_idx, Wq, Wk, Wv, Wp):
    raise NotImplementedError("write your pallas kernel here")


# speedup vs baseline: 21.9089x; 21.9089x over previous
import functools
import math

import jax
import jax.numpy as jnp
from jax import lax
from jax.experimental import pallas as pl
from jax.experimental.pallas import tpu as pltpu
from jax.experimental.pallas import tpu_sc as plsc

Na = 10000
E = 160000
C = 256
D = 256
H = 8
DH = D // H
HD2 = D // 2
HP = 4
INV_SQRT_DH = 1.0 / math.sqrt(DH)

NW = 32
NAP = 10240
EP = 163840
EPT = EP // NW
CH = 1280
NCH = EPT // CH
GB = 32
NBLK = CH // GB
RWB = NAP // 16


def _proj_body(fa, fb, wq, wk, wv, q0, q1, k0, k1, v0, v1):
    q = jnp.dot(fa[...], wq[...], preferred_element_type=jnp.float32)
    k = jnp.dot(fb[...], wk[...], preferred_element_type=jnp.float32)
    v = jnp.dot(fb[...], wv[...], preferred_element_type=jnp.float32)
    q0[...] = q[:, :HD2]
    q1[...] = q[:, HD2:]
    k0[...] = k[:, :HD2]
    k1[...] = k[:, HD2:]
    v0[...] = v[:, :HD2]
    v1[...] = v[:, HD2:]


def _proj(Fa, Fb, Wq, Wk, Wv):
    blk = 1000
    grid = Na // blk
    row_spec = pl.BlockSpec((blk, C), lambda i: (i, 0))
    w_spec = pl.BlockSpec((C, D), lambda i: (0, 0))
    half_spec = pl.BlockSpec((blk, HD2), lambda i: (i, 0))
    half = jax.ShapeDtypeStruct((Na, HD2), jnp.float32)
    return pl.pallas_call(
        _proj_body,
        grid=(grid,),
        in_specs=[row_spec, row_spec, w_spec, w_spec, w_spec],
        out_specs=[half_spec] * 6,
        out_shape=[half] * 6,
    )(Fa, Fb, Wq, Wk, Wv)


def _edge_body(q0, q1, k0, k1, v0, v1, ag_hbm, as_hbm, b_hbm, oh_hbm,
               val0, val1, dval,
               ag_v, as_v, b_v, as_blk, r0, r1, r2, r3, contrib,
               oh_v, zrow, shared, sem):
    core = lax.axis_index("c")
    sid = lax.axis_index("s")
    wid = sid * 2 + core

    pltpu.sync_copy(oh_hbm, oh_v)

    zf = jnp.zeros((16,), jnp.float32)
    for r in range(16):
        for cc in range(HD2 // 16):
            zrow[r, pl.ds(cc * 16, 16)] = zf

    def _zero_shared():
        for z in range(RWB // 16):
            pltpu.sync_copy(zrow, shared.at[pl.ds(sid * RWB + z * 16, 16)])

    def _load_chunk(ci):
        base = wid * EPT + ci * CH
        pltpu.sync_copy(ag_hbm.at[pl.ds(base, CH)], ag_v)
        pltpu.sync_copy(as_hbm.at[pl.ds(base, CH)], as_v)
        pltpu.sync_copy(b_hbm.at[pl.ds(base, CH)], b_v)

    for p, (qh, kh, vh, val) in enumerate(((q0, k0, v0, val0),
                                           (q1, k1, v1, val1))):
        _zero_shared()
        plsc.subcore_barrier()

        def _chunk(ci, _, qh=qh, kh=kh, vh=vh):
            _load_chunk(ci)

            def _block(blk, _b, qh=qh, kh=kh, vh=vh):
                base = blk * GB
                cq = pltpu.async_copy(qh.at[ag_v.at[pl.ds(base, GB)]], r0, sem)
                ck = pltpu.async_copy(kh.at[b_v.at[pl.ds(base, GB)]], r1, sem)
                cv = pltpu.async_copy(vh.at[b_v.at[pl.ds(base, GB)]], r2, sem)
                for t in range(GB // 16):
                    as_blk[pl.ds(t * 16, 16)] = as_v[pl.ds(base + t * 16, 16)]
                cq.wait()
                ck.wait()
                cv.wait()

                def _grp(g, _2):
                    for lane in range(16):
                        j = g * 16 + lane
                        lv = None
                        for h in range(HP):
                            t = (r0[j, pl.ds(h * DH, 16)]
                                 * r1[j, pl.ds(h * DH, 16)]
                                 + r0[j, pl.ds(h * DH + 16, 16)]
                                 * r1[j, pl.ds(h * DH + 16, 16)])
                            term = oh_v[h] * (jnp.sum(t) * INV_SQRT_DH)
                            lv = term if lv is None else lv + term
                        ev = jnp.exp(lv)
                        for h in range(HP):
                            eh = ev[h]
                            contrib[j, pl.ds(h * DH, 16)] = (
                                r2[j, pl.ds(h * DH, 16)] * eh)
                            contrib[j, pl.ds(h * DH + 16, 16)] = (
                                r2[j, pl.ds(h * DH + 16, 16)] * eh)
                    return _2
                lax.fori_loop(0, GB // 16, _grp, None)
                pltpu.sync_copy(contrib, shared.at[as_blk], add=True)
                return _b
            lax.fori_loop(0, NBLK, _block, None)
            return _
        lax.fori_loop(0, NCH, _chunk, None)
        plsc.subcore_barrier()
        pltpu.sync_copy(shared.at[pl.ds(sid * RWB, RWB)],
                        val.at[pl.ds(core * NAP + sid * RWB, RWB)])
        plsc.subcore_barrier()

    _zero_shared()
    plsc.subcore_barrier()

    def _dchunk(ci, _):
        _load_chunk(ci)

        def _dblock(blk, _b):
            base = blk * GB
            c0 = pltpu.async_copy(q0.at[ag_v.at[pl.ds(base, GB)]], r0, sem)
            c1 = pltpu.async_copy(q1.at[ag_v.at[pl.ds(base, GB)]], r1, sem)
            c2 = pltpu.async_copy(k0.at[b_v.at[pl.ds(base, GB)]], r2, sem)
            c3 = pltpu.async_copy(k1.at[b_v.at[pl.ds(base, GB)]], r3, sem)
            for t in range(GB // 16):
                as_blk[pl.ds(t * 16, 16)] = as_v[pl.ds(base + t * 16, 16)]
            c0.wait()
            c1.wait()
            c2.wait()
            c3.wait()

            def _grp(g, _2):
                ones_v = oh_v[16]
                for lane in range(16):
                    j = g * 16 + lane
                    lv = None
                    for h8 in range(H):
                        qr = r0 if h8 < HP else r1
                        kr = r2 if h8 < HP else r3
                        off = (h8 % HP) * DH
                        t = (qr[j, pl.ds(off, 16)] * kr[j, pl.ds(off, 16)]
                             + qr[j, pl.ds(off + 16, 16)]
                             * kr[j, pl.ds(off + 16, 16)])
                        term = oh_v[h8] * (jnp.sum(t) * INV_SQRT_DH)
                        lv = term if lv is None else lv + term
                    ev = jnp.exp(lv)
                    for h8 in range(H):
                        contrib[j, pl.ds(h8 * 16, 16)] = ones_v * ev[h8]
                return _2
            lax.fori_loop(0, GB // 16, _grp, None)
            pltpu.sync_copy(contrib, shared.at[as_blk], add=True)
            return _b
        lax.fori_loop(0, NBLK, _dblock, None)
        return _
    lax.fori_loop(0, NCH, _dchunk, None)
    plsc.subcore_barrier()
    pltpu.sync_copy(shared.at[pl.ds(sid * RWB, RWB)],
                    dval.at[pl.ds(core * NAP + sid * RWB, RWB)])


def _edge_pass(q0, q1, k0, k1, v0, v1, ag, as_, bg, ohtab):
    mesh = plsc.VectorSubcoreMesh(core_axis_name="c", subcore_axis_name="s")
    out = pltpu.HBM((2 * NAP, HD2), jnp.float32)
    kern = functools.partial(
        pl.kernel, mesh=mesh,
        compiler_params=pltpu.CompilerParams(needs_layout_passes=False),
        out_type=[out, out, out],
        scratch_types=[
            pltpu.VMEM((CH,), jnp.int32),
            pltpu.VMEM((CH,), jnp.int32),
            pltpu.VMEM((CH,), jnp.int32),
            pltpu.VMEM((GB,), jnp.int32),
            pltpu.VMEM((GB, HD2), jnp.float32),
            pltpu.VMEM((GB, HD2), jnp.float32),
            pltpu.VMEM((GB, HD2), jnp.float32),
            pltpu.VMEM((GB, HD2), jnp.float32),
            pltpu.VMEM((GB, HD2), jnp.float32),
            pltpu.VMEM((24, 16), jnp.float32),
            pltpu.VMEM((16, HD2), jnp.float32),
            pltpu.VMEM_SHARED((NAP, HD2), jnp.float32),
            pltpu.SemaphoreType.DMA,
        ],
    )(_edge_body)
    return kern(q0, q1, k0, k1, v0, v1, ag, as_, bg, ohtab)


def _final_body(v00, v01, v10, v11, d00, d01, fa, sm, wp, out):
    num = jnp.concatenate([v00[...] + v01[...], v10[...] + v11[...]], axis=1)
    dex = jnp.dot(d00[...] + d01[...], sm[...],
                  preferred_element_type=jnp.float32)
    w = num / jnp.where(dex > 0.0, dex, 1.0)
    out[...] = fa[...] + jnp.dot(w, wp[...], preferred_element_type=jnp.float32)


def _final(v00, v01, v10, v11, d00, d01, Fap, Wp):
    rows = jnp.arange(HD2)
    cols = jnp.arange(C)
    sm = ((rows[:, None] % 16 == 0)
          & (rows[:, None] // 16 == cols[None, :] // DH)).astype(jnp.float32)
    blk = 1280
    grid = NAP // blk
    r_spec = pl.BlockSpec((blk, HD2), lambda i: (i, 0))
    return pl.pallas_call(
        _final_body,
        grid=(grid,),
        in_specs=[r_spec, r_spec, r_spec, r_spec, r_spec, r_spec,
                  pl.BlockSpec((blk, C), lambda i: (i, 0)),
                  pl.BlockSpec((HD2, C), lambda i: (0, 0)),
                  pl.BlockSpec((D, C), lambda i: (0, 0))],
        out_specs=pl.BlockSpec((blk, C), lambda i: (i, 0)),
        out_shape=jax.ShapeDtypeStruct((NAP, C), jnp.float32),
    )(v00, v01, v10, v11, d00, d01, Fap, sm, Wp)


def kernel(Fa, Fb, a_idx, b_idx, Wq, Wk, Wv, Wp):
    q0, q1, k0, k1, v0, v1 = _proj(Fa, Fb, Wq, Wk, Wv)
    a32 = a_idx.astype(jnp.int32)
    b32 = b_idx.astype(jnp.int32)
    pad = EP - E
    ag = jnp.concatenate([a32, jnp.zeros((pad,), jnp.int32)])
    as_ = jnp.concatenate([a32, jnp.full((pad,), NAP - 1, jnp.int32)])
    bg = jnp.concatenate([b32, jnp.zeros((pad,), jnp.int32)])
    eye = (jnp.arange(16)[None, :] == jnp.arange(16)[:, None]).astype(jnp.float32)
    ohtab = jnp.concatenate([eye, jnp.ones((1, 16), jnp.float32),
                             jnp.zeros((7, 16), jnp.float32)])
    val0, val1, dval = _edge_pass(q0, q1, k0, k1, v0, v1, ag, as_, bg, ohtab)
    Fap = jnp.concatenate([Fa, jnp.zeros((NAP - Na, C), jnp.float32)])
    out = _final(val0[:NAP], val0[NAP:], val1[:NAP], val1[NAP:],
                 dval[:NAP], dval[NAP:], Fap, Wp)
    return out[:Na]
